# baseline (speedup 1.0000x reference)
import functools
import jax
import jax.numpy as jnp
from jax import lax
from jax.experimental import pallas as pl
from jax.experimental.pallas import tpu as pltpu
from jax.experimental.pallas import tpu_sc as plsc

_B = 128
_NC = 1
_NS = 16
_NW = _NC * _NS


def _sc_body(n_per_w, table_hbm, out_hbm, slice_v, sem):
    wid = lax.axis_index("s") * _NC + lax.axis_index("c")
    base = wid * n_per_w
    pltpu.sync_copy(table_hbm.at[pl.ds(base, n_per_w)], slice_v)
    copies = [
        pltpu.make_async_copy(
            slice_v, out_hbm.at[b, pl.ds(base, n_per_w)], sem
        )
        for b in range(_B)
    ]
    for c in copies:
        c.start()
    for c in copies:
        c.wait()


def kernel(batch_size, table):
    n, d = table.shape
    n_per_w = n // _NW
    mesh = plsc.VectorSubcoreMesh(
        core_axis_name="c", subcore_axis_name="s", num_cores=1
    )
    k = pl.kernel(
        functools.partial(_sc_body, n_per_w),
        out_type=jax.ShapeDtypeStruct((_B, n, d), table.dtype),
        mesh=mesh,
        scratch_types=[
            pltpu.VMEM((n_per_w, d), table.dtype),
            pltpu.SemaphoreType.DMA,
        ],
        compiler_params=pltpu.CompilerParams(use_tc_tiling_on_sc=True),
    )
    return k(table)


# speedup vs baseline: 1.2446x; 1.2446x over previous
import functools
import jax
import jax.numpy as jnp
from jax import lax
from jax.experimental import pallas as pl
from jax.experimental.pallas import tpu as pltpu
from jax.experimental.pallas import tpu_sc as plsc

_B = 128
_NC = 2
_NS = 16
_NW = _NC * _NS
_GRP = 8


def _sc_body(n_per_w, table_hbm, out_hbm, slice_v, sem):
    wid = lax.axis_index("s") * _NC + lax.axis_index("c")
    base = wid * n_per_w
    pltpu.sync_copy(table_hbm.at[pl.ds(base, n_per_w)], slice_v)

    def group(g, carry):
        b0 = g * _GRP
        for j in range(_GRP):
            pltpu.make_async_copy(
                slice_v, out_hbm.at[b0 + j, pl.ds(base, n_per_w)], sem
            ).start()
        for j in range(_GRP):
            pltpu.make_async_copy(
                slice_v, out_hbm.at[b0 + j, pl.ds(base, n_per_w)], sem
            ).wait()
        return carry

    lax.fori_loop(0, _B // _GRP, group, 0)


def kernel(batch_size, table):
    n, d = table.shape
    n_per_w = n // _NW
    mesh = plsc.VectorSubcoreMesh(core_axis_name="c", subcore_axis_name="s")
    k = pl.kernel(
        functools.partial(_sc_body, n_per_w),
        out_type=jax.ShapeDtypeStruct((_B, n, d), table.dtype),
        mesh=mesh,
        scratch_types=[
            pltpu.VMEM((n_per_w, d), table.dtype),
            pltpu.SemaphoreType.DMA,
        ],
    )
    return k(table)
